# baseline (speedup 1.0000x reference)
import functools

import jax
import jax.numpy as jnp
from jax import lax
from jax.experimental import pallas as pl
from jax.experimental.pallas import tpu as pltpu
from jax.experimental.pallas import tpu_sc as plsc

N_SPINS = 20
BATCH = 16384
NUM_WORKERS = 32
B_PER_W = BATCH // NUM_WORKERS
ROWS = 4


def _sc_body(sig_hbm, amps_hbm, out_hbm, sig_v, idx_v, out_v, sem, gsem):
    nc = 2
    wid = lax.axis_index("s") * nc + lax.axis_index("c")
    base = wid * B_PER_W

    pltpu.sync_copy(sig_hbm.at[:, pl.ds(base, B_PER_W)], sig_v)

    zeros = jnp.zeros((16,), jnp.int32)
    ones = jnp.ones((16,), jnp.int32)

    def group_body(g, _):
        c0 = g * 16

        def spin_body(i, acc):
            v = sig_v[i, pl.ds(c0, 16)]
            return acc | jnp.where(v > 0.0, ones << i, zeros)

        idx_v[pl.ds(c0, 16)] = lax.fori_loop(0, N_SPINS, spin_body, zeros,
                                             unroll=False)
        return 0

    lax.fori_loop(0, B_PER_W // 16, group_body, 0, unroll=False)

    gathers = [
        pltpu.async_copy(amps_hbm.at[idx_v.at[pl.ds(r * 128, 128)]],
                         out_v.at[pl.ds(r * 128, 128)], gsem)
        for r in range(ROWS)
    ]
    for cp in gathers:
        cp.wait()

    pltpu.sync_copy(out_v, out_hbm.at[pl.ds(base, B_PER_W)])


@jax.jit
def kernel(sigma, amps):
    sig_t = sigma.T
    mesh = plsc.VectorSubcoreMesh(core_axis_name="c", subcore_axis_name="s")
    k = functools.partial(
        pl.kernel,
        mesh=mesh,
        out_type=jax.ShapeDtypeStruct((BATCH,), jnp.float32),
        scratch_types=[
            pltpu.VMEM((N_SPINS, B_PER_W), jnp.float32),
            pltpu.VMEM((B_PER_W,), jnp.int32),
            pltpu.VMEM((B_PER_W,), jnp.float32),
            pltpu.SemaphoreType.DMA,
            pltpu.SemaphoreType.DMA,
        ],
        compiler_params=pltpu.CompilerParams(needs_layout_passes=False),
    )(_sc_body)
    return k(sig_t, amps)


# speedup vs baseline: 1.1042x; 1.1042x over previous
import functools

import jax
import jax.numpy as jnp
from jax import lax
from jax.experimental import pallas as pl
from jax.experimental.pallas import tpu as pltpu
from jax.experimental.pallas import tpu_sc as plsc

N_SPINS = 20
BATCH = 16384
NUM_WORKERS = 32
B_PER_W = BATCH // NUM_WORKERS
ROWS = 4


def _sc_body(sig_hbm, amps_hbm, out_hbm, sig_v, idx_v, out_v,
             sem0, sem1, gsem):
    nc = 2
    wid = lax.axis_index("s") * nc + lax.axis_index("c")
    base = wid * B_PER_W
    half = B_PER_W // 2

    loads = [
        pltpu.async_copy(sig_hbm.at[:, pl.ds(base + h * half, half)],
                         sig_v.at[:, pl.ds(h * half, half)], s)
        for h, s in ((0, sem0), (1, sem1))
    ]

    zeros = jnp.zeros((16,), jnp.int32)
    gathers = []
    for h in range(2):
        loads[h].wait()

        def group_body(g, _, h=h):
            c0 = h * half + g * 16
            acc = zeros
            for i in range(N_SPINS):
                v = sig_v[i, pl.ds(c0, 16)]
                acc = acc | jnp.where(v > 0.0,
                                      jnp.full((16,), 1 << i, jnp.int32),
                                      zeros)
            idx_v[pl.ds(c0, 16)] = acc
            return 0

        lax.fori_loop(0, half // 16, group_body, 0, unroll=False)
        for r in (2 * h, 2 * h + 1):
            gathers.append(
                pltpu.async_copy(amps_hbm.at[idx_v.at[pl.ds(r * 128, 128)]],
                                 out_v.at[pl.ds(r * 128, 128)], gsem))
    for cp in gathers:
        cp.wait()

    pltpu.sync_copy(out_v, out_hbm.at[pl.ds(base, B_PER_W)])


@jax.jit
def kernel(sigma, amps):
    sig_t = sigma.T
    mesh = plsc.VectorSubcoreMesh(core_axis_name="c", subcore_axis_name="s")
    k = functools.partial(
        pl.kernel,
        mesh=mesh,
        out_type=jax.ShapeDtypeStruct((BATCH,), jnp.float32),
        scratch_types=[
            pltpu.VMEM((N_SPINS, B_PER_W), jnp.float32),
            pltpu.VMEM((B_PER_W,), jnp.int32),
            pltpu.VMEM((B_PER_W,), jnp.float32),
            pltpu.SemaphoreType.DMA,
            pltpu.SemaphoreType.DMA,
            pltpu.SemaphoreType.DMA,
        ],
        compiler_params=pltpu.CompilerParams(needs_layout_passes=False),
    )(_sc_body)
    return k(sig_t, amps)


# speedup vs baseline: 1.1677x; 1.0575x over previous
import functools

import jax
import jax.numpy as jnp
from jax import lax
from jax.experimental import pallas as pl
from jax.experimental.pallas import tpu as pltpu
from jax.experimental.pallas import tpu_sc as plsc

N_SPINS = 20
BATCH = 16384
NUM_WORKERS = 32
B_PER_W = BATCH // NUM_WORKERS
ROWS = 4


def _tc_bitpack(sig_ref, idx_ref):
    s = sig_ref[...]
    pw = jnp.int32(1) << lax.broadcasted_iota(jnp.int32, (N_SPINS, 1), 0)
    bits = jnp.where(s > 0.0, pw, jnp.int32(0))
    idx_ref[...] = jnp.sum(bits, axis=0)


def _sc_gather(idx_hbm, amps_hbm, out_hbm, idx_v, out_v, gsem):
    nc = 2
    wid = lax.axis_index("s") * nc + lax.axis_index("c")
    base = wid * B_PER_W

    pltpu.sync_copy(idx_hbm.at[pl.ds(base, B_PER_W)], idx_v)

    gathers = [
        pltpu.async_copy(amps_hbm.at[idx_v.at[pl.ds(r * 128, 128)]],
                         out_v.at[pl.ds(r * 128, 128)], gsem)
        for r in range(ROWS)
    ]
    for cp in gathers:
        cp.wait()

    pltpu.sync_copy(out_v, out_hbm.at[pl.ds(base, B_PER_W)])


@jax.jit
def kernel(sigma, amps):
    sig_t = sigma.T
    idx = pl.pallas_call(
        _tc_bitpack,
        out_shape=jax.ShapeDtypeStruct((BATCH,), jnp.int32),
    )(sig_t)

    mesh = plsc.VectorSubcoreMesh(core_axis_name="c", subcore_axis_name="s")
    k = functools.partial(
        pl.kernel,
        mesh=mesh,
        out_type=jax.ShapeDtypeStruct((BATCH,), jnp.float32),
        scratch_types=[
            pltpu.VMEM((B_PER_W,), jnp.int32),
            pltpu.VMEM((B_PER_W,), jnp.float32),
            pltpu.SemaphoreType.DMA,
        ],
        compiler_params=pltpu.CompilerParams(needs_layout_passes=False),
    )(_sc_gather)
    return k(idx, amps)
